# Initial kernel scaffold; baseline (speedup 1.0000x reference)
import jax
import jax.numpy as jnp
from jax.experimental import pallas as pl


def kernel(hidden_states, ln1_w, ln2_w, Wq, Wk, Wv, Wo, Wr, Wgate, Wup, Wdown):
    raise NotImplementedError("write your pallas kernel here")


# speedup vs baseline: 2.6513x; 2.6513x over previous
import jax
import jax.numpy as jnp
import numpy as np
from jax.experimental import pallas as pl
from jax.experimental.pallas import tpu as pltpu

B, S, D = 1, 2048, 1024
H, HD = 16, 64
E, K, F = 64, 2, 2048
EPS = 1e-6
THETA = 10000.0
NEG = -1e9

SBLK = 256
NI = S // SBLK
TT = 128
G = (S * K) // TT + E
FB = 1024
NF = F // FB


def _rope(x, cos, sin):
    pieces = []
    for c in range(H):
        a = x[:, c * HD: c * HD + HD // 2]
        b = x[:, c * HD + HD // 2: (c + 1) * HD]
        pieces.append(-b)
        pieces.append(a)
    rot = jnp.concatenate(pieces, axis=1)
    return x * cos + rot * sin


def _qkv_kernel(x_ref, w_ref, lnw_ref, cos_ref, sin_ref, out_ref):
    x = x_ref[...]
    v = jnp.mean(x * x, axis=-1, keepdims=True)
    xn = x * jax.lax.rsqrt(v + EPS) * lnw_ref[...]
    y = jnp.dot(xn, w_ref[...], preferred_element_type=jnp.float32)
    cos = cos_ref[...]
    sin = sin_ref[...]
    q = _rope(y[:, :D], cos, sin)
    k = _rope(y[:, D:2 * D], cos, sin)
    out_ref[...] = jnp.concatenate([q, k, y[:, 2 * D:]], axis=1)


def _attn_kernel(q_ref, k_ref, v_ref, o_ref):
    i = pl.program_id(1)
    q = q_ref[0]
    s = jax.lax.dot_general(q, k_ref[0], (((1,), (1,)), ((), ())),
                            preferred_element_type=jnp.float32)
    s = s * (1.0 / np.sqrt(HD))
    rows = i * SBLK + jax.lax.broadcasted_iota(jnp.int32, (SBLK, S), 0)
    cols = jax.lax.broadcasted_iota(jnp.int32, (SBLK, S), 1)
    s = s + jnp.where(cols <= rows, 0.0, NEG)
    m = jnp.max(s, axis=1, keepdims=True)
    p = jnp.exp(s - m)
    p = p / jnp.sum(p, axis=1, keepdims=True)
    o_ref[0] = jnp.dot(p, v_ref[0], preferred_element_type=jnp.float32)


def _oproj_kernel(o_ref, res_ref, wo_ref, ln2_ref, wr_ref,
                  h_ref, hn_ref, lg_ref, i1_ref, i2_ref, w1_ref, w2_ref):
    h = jnp.dot(o_ref[...], wo_ref[...],
                preferred_element_type=jnp.float32) + res_ref[...]
    h_ref[...] = h
    v = jnp.mean(h * h, axis=-1, keepdims=True)
    hn = h * jax.lax.rsqrt(v + EPS) * ln2_ref[...]
    hn_ref[...] = hn
    lg = jnp.dot(hn, wr_ref[...], preferred_element_type=jnp.float32)
    lg_ref[...] = lg
    mx = jnp.max(lg, axis=1, keepdims=True)
    ex = jnp.exp(lg - mx)
    pr = ex / jnp.sum(ex, axis=1, keepdims=True)
    lane = jax.lax.broadcasted_iota(jnp.int32, (SBLK, E), 1)
    m1 = jnp.max(pr, axis=1, keepdims=True)
    i1 = jnp.min(jnp.where(pr == m1, lane, E), axis=1, keepdims=True)
    pr2 = jnp.where(lane == i1, -1.0, pr)
    m2 = jnp.max(pr2, axis=1, keepdims=True)
    i2 = jnp.min(jnp.where(pr2 == m2, lane, E), axis=1, keepdims=True)
    ssum = m1 + m2
    i1_ref[...] = i1
    i2_ref[...] = i2
    w1_ref[...] = m1 / ssum
    w2_ref[...] = m2 / ssum


def _moe_kernel(eid_ref, cnt_ref,
                hn_ref, res_ref, tokc_ref, tokr_ref, wt_ref,
                wg_ref, wu_ref, wd_ref,
                out_ref, x_sc, y_sc):
    t = pl.program_id(0)
    f = pl.program_id(1)

    @pl.when((t == 0) & (f == 0))
    def _init():
        out_ref[...] = res_ref[...]

    @pl.when(cnt_ref[t] > 0)
    def _body():
        tokc = tokc_ref[0]

        @pl.when(f == 0)
        def _gather():
            iot = jax.lax.broadcasted_iota(jnp.int32, (TT, S), 1)
            oh = (iot == tokc).astype(jnp.float32)
            x_sc[...] = jnp.dot(oh, hn_ref[...],
                                preferred_element_type=jnp.float32)

        x = x_sc[...]
        g = jnp.dot(x, wg_ref[0], preferred_element_type=jnp.float32)
        u = jnp.dot(x, wu_ref[0], preferred_element_type=jnp.float32)
        p = g * jax.lax.logistic(g) * u
        yp = jnp.dot(p, wd_ref[0], preferred_element_type=jnp.float32)

        @pl.when(f == 0)
        def _acc0():
            y_sc[...] = yp

        @pl.when(f != 0)
        def _acc1():
            y_sc[...] += yp

        @pl.when(f == NF - 1)
        def _scatter():
            yw = y_sc[...] * wt_ref[0]
            rowio = jax.lax.broadcasted_iota(jnp.int32, (S, TT), 0)
            oht = (rowio == tokr_ref[0]).astype(jnp.float32)
            out_ref[...] += jnp.dot(oht, yw,
                                    preferred_element_type=jnp.float32)


def kernel(hidden_states, ln1_w, ln2_w, Wq, Wk, Wv, Wo, Wr, Wgate, Wup, Wdown):
    f32 = jnp.float32
    x = hidden_states.reshape(S, D)
    Wqkv = jnp.concatenate([Wq, Wk, Wv], axis=1)

    inv_freq = 1.0 / (THETA ** (jnp.arange(0, HD, 2, dtype=f32) / HD))
    t = jnp.arange(S, dtype=f32)
    freqs = jnp.outer(t, inv_freq)
    emb = jnp.concatenate([freqs, freqs], axis=-1)
    cos_t = jnp.tile(jnp.cos(emb), (1, H))
    sin_t = jnp.tile(jnp.sin(emb), (1, H))

    qkv = pl.pallas_call(
        _qkv_kernel,
        grid=(NI,),
        in_specs=[
            pl.BlockSpec((SBLK, D), lambda i: (i, 0)),
            pl.BlockSpec((D, 3 * D), lambda i: (0, 0)),
            pl.BlockSpec((1, D), lambda i: (0, 0)),
            pl.BlockSpec((SBLK, D), lambda i: (i, 0)),
            pl.BlockSpec((SBLK, D), lambda i: (i, 0)),
        ],
        out_specs=pl.BlockSpec((SBLK, 3 * D), lambda i: (i, 0)),
        out_shape=jax.ShapeDtypeStruct((S, 3 * D), f32),
    )(x, Wqkv, ln1_w.reshape(1, D), cos_t, sin_t)

    qkv3 = qkv.reshape(S, 3 * H, HD).transpose(1, 0, 2)

    o3 = pl.pallas_call(
        _attn_kernel,
        grid=(H, NI),
        in_specs=[
            pl.BlockSpec((1, SBLK, HD), lambda h, i: (h, i, 0)),
            pl.BlockSpec((1, S, HD), lambda h, i: (H + h, 0, 0)),
            pl.BlockSpec((1, S, HD), lambda h, i: (2 * H + h, 0, 0)),
        ],
        out_specs=pl.BlockSpec((1, SBLK, HD), lambda h, i: (h, i, 0)),
        out_shape=jax.ShapeDtypeStruct((H, S, HD), f32),
    )(qkv3, qkv3, qkv3)
    o = o3.transpose(1, 0, 2).reshape(S, D)

    h, hn, logits, i1, i2, w1, w2 = pl.pallas_call(
        _oproj_kernel,
        grid=(NI,),
        in_specs=[
            pl.BlockSpec((SBLK, D), lambda i: (i, 0)),
            pl.BlockSpec((SBLK, D), lambda i: (i, 0)),
            pl.BlockSpec((D, D), lambda i: (0, 0)),
            pl.BlockSpec((1, D), lambda i: (0, 0)),
            pl.BlockSpec((D, E), lambda i: (0, 0)),
        ],
        out_specs=[
            pl.BlockSpec((SBLK, D), lambda i: (i, 0)),
            pl.BlockSpec((SBLK, D), lambda i: (i, 0)),
            pl.BlockSpec((SBLK, E), lambda i: (i, 0)),
            pl.BlockSpec((SBLK, 1), lambda i: (i, 0)),
            pl.BlockSpec((SBLK, 1), lambda i: (i, 0)),
            pl.BlockSpec((SBLK, 1), lambda i: (i, 0)),
            pl.BlockSpec((SBLK, 1), lambda i: (i, 0)),
        ],
        out_shape=[
            jax.ShapeDtypeStruct((S, D), f32),
            jax.ShapeDtypeStruct((S, D), f32),
            jax.ShapeDtypeStruct((S, E), f32),
            jax.ShapeDtypeStruct((S, 1), jnp.int32),
            jax.ShapeDtypeStruct((S, 1), jnp.int32),
            jax.ShapeDtypeStruct((S, 1), f32),
            jax.ShapeDtypeStruct((S, 1), f32),
        ],
    )(o, x, Wo, ln2_w.reshape(1, D), Wr)

    experts = jnp.concatenate([i1[:, 0], i2[:, 0]])
    tokens = jnp.concatenate([jnp.arange(S, dtype=jnp.int32)] * 2)
    weights = jnp.concatenate([w1[:, 0], w2[:, 0]])
    order = jnp.argsort(experts)
    st = tokens[order]
    sw = weights[order]
    counts = jnp.zeros((E,), jnp.int32).at[experts].add(1)
    cum = jnp.cumsum(counts)
    offsets = cum - counts
    nt = (counts + TT - 1) // TT
    cumt = jnp.cumsum(nt)
    t_ar = jnp.arange(G, dtype=jnp.int32)
    e_of_t = jnp.searchsorted(cumt, t_ar, side='right').astype(jnp.int32)
    e_of_t = jnp.minimum(e_of_t, E - 1)
    local = t_ar - (cumt - nt)[e_of_t]
    cnt_t = jnp.clip(counts[e_of_t] - local * TT, 0, TT).astype(jnp.int32)
    eids = jax.lax.cummax(jnp.where(cnt_t > 0, e_of_t, 0))
    start = offsets[e_of_t] + local * TT
    idxs = start[:, None] + jnp.arange(TT, dtype=jnp.int32)[None]
    validm = jnp.arange(TT, dtype=jnp.int32)[None] < cnt_t[:, None]
    idxc = jnp.clip(idxs, 0, S * K - 1)
    tok_tab = jnp.where(validm, st[idxc], 0).astype(jnp.int32)
    w_tab = jnp.where(validm, sw[idxc], 0.0).astype(f32)

    tokc = tok_tab.reshape(G, TT, 1)
    tokr = tok_tab.reshape(G, 1, TT)
    wt = w_tab.reshape(G, TT, 1)

    out = pl.pallas_call(
        _moe_kernel,
        grid_spec=pltpu.PrefetchScalarGridSpec(
            num_scalar_prefetch=2,
            grid=(G, NF),
            in_specs=[
                pl.BlockSpec((S, D), lambda t, f, e, c: (0, 0)),
                pl.BlockSpec((S, D), lambda t, f, e, c: (0, 0)),
                pl.BlockSpec((1, TT, 1), lambda t, f, e, c: (t, 0, 0)),
                pl.BlockSpec((1, 1, TT), lambda t, f, e, c: (t, 0, 0)),
                pl.BlockSpec((1, TT, 1), lambda t, f, e, c: (t, 0, 0)),
                pl.BlockSpec((1, D, FB), lambda t, f, e, c: (e[t], 0, f)),
                pl.BlockSpec((1, D, FB), lambda t, f, e, c: (e[t], 0, f)),
                pl.BlockSpec((1, FB, D), lambda t, f, e, c: (e[t], f, 0)),
            ],
            out_specs=pl.BlockSpec((S, D), lambda t, f, e, c: (0, 0)),
            scratch_shapes=[
                pltpu.VMEM((TT, D), f32),
                pltpu.VMEM((TT, D), f32),
            ],
        ),
        out_shape=jax.ShapeDtypeStruct((S, D), f32),
    )(eids, cnt_t, hn, h, tokc, tokr, wt, Wgate, Wup, Wdown)

    return (out.reshape(B, S, D), logits)
